# baseline (speedup 1.0000x reference)
import functools

import jax
import jax.numpy as jnp
from jax import lax
from jax.experimental import pallas as pl
from jax.experimental.pallas import tpu as pltpu
from jax.experimental.pallas import tpu_sc as plsc

_NC = 2
_NS = 16
_NW = _NC * _NS
_NBUF = 4
_GLA = 2
_L = 16


@jax.jit
def _gather_rows(tensor, wpad):
    nb, hist = tensor.shape
    dimp = wpad.shape[1]
    dim = dimp // 2
    bpw = nb // _NW
    mesh = plsc.VectorSubcoreMesh(core_axis_name="c", subcore_axis_name="s")

    @functools.partial(
        pl.kernel,
        mesh=mesh,
        out_type=jax.ShapeDtypeStruct((nb, hist, dim), jnp.float32),
        scratch_types=[
            pltpu.VMEM((bpw, hist), jnp.int32),
            *[pltpu.VMEM((hist, dimp), jnp.float32) for _ in range(_NBUF)],
            *[pltpu.VMEM((hist, dim), jnp.float32) for _ in range(_NBUF)],
            *[pltpu.SemaphoreType.DMA for _ in range(1 + 2 * _NBUF)],
        ],
    )
    def k(idx_hbm, table_hbm, out_hbm, idx_v, *bufs_sems):
        wide = bufs_sems[:_NBUF]
        narrow = bufs_sems[_NBUF:2 * _NBUF]
        isem = bufs_sems[2 * _NBUF]
        gsem = bufs_sems[2 * _NBUF + 1:2 * _NBUF + 1 + _NBUF]
        osem = bufs_sems[2 * _NBUF + 1 + _NBUF:]
        wid = lax.axis_index("s") * _NC + lax.axis_index("c")
        base = wid * bpw
        pltpu.async_copy(idx_hbm.at[pl.ds(base, bpw)], idx_v, isem).wait()

        def fire_gather(b):
            r = b % _NBUF
            return pltpu.async_copy(table_hbm.at[idx_v.at[b]], wide[r], gsem[r])

        def compact(r):
            w = wide[r]
            nr = narrow[r]

            def row(r2, _):
                for gq in range(dim // _L):
                    nr[r2, pl.ds(gq * _L, _L)] = w[r2, pl.ds(gq * _L, _L)]
                return 0

            lax.fori_loop(0, hist, row, 0, unroll=2)

        g = {}
        o = {}
        for b in range(min(_GLA, bpw)):
            g[b] = fire_gather(b)
        for b in range(bpw):
            r = b % _NBUF
            nb_ = b + _GLA
            if nb_ < bpw:
                if nb_ >= _NBUF:
                    o[nb_ - _NBUF].wait()
                g[nb_] = fire_gather(nb_)
            g[b].wait()
            compact(r)
            o[b] = pltpu.async_copy(narrow[r], out_hbm.at[base + b], osem[r])
        for b in range(max(0, bpw - _NBUF), bpw):
            o[b].wait()

    return k(tensor, wpad)


def kernel(tensor, weight):
    wpad = jnp.pad(weight, ((0, 0), (0, weight.shape[1])))
    return _gather_rows(tensor.astype(jnp.int32), wpad)


# speedup vs baseline: 1.0244x; 1.0244x over previous
import functools

import jax
import jax.numpy as jnp
from jax import lax
from jax.experimental import pallas as pl
from jax.experimental.pallas import tpu as pltpu
from jax.experimental.pallas import tpu_sc as plsc

_NC = 2
_NS = 16
_NW = _NC * _NS
_L = 16


@functools.partial(jax.jit, static_argnames=("nb", "hist", "dim"))
def _gather_rows(idx_flat, table_flat, nb, hist, dim):
    vocab_words = table_flat.shape[0]
    n_dt = dim // 8
    n_hc = (hist + 3) // 4
    mesh = plsc.VectorSubcoreMesh(core_axis_name="c", subcore_axis_name="s")

    @functools.partial(
        pl.kernel,
        mesh=mesh,
        compiler_params=pltpu.CompilerParams(needs_layout_passes=False),
        out_type=jax.ShapeDtypeStruct((hist, dim, nb), jnp.float32),
        scratch_types=[
            pltpu.VMEM((vocab_words,), jnp.float32),
            pltpu.VMEM((nb,), jnp.int32),
            pltpu.VMEM((nb,), jnp.int32),
            pltpu.VMEM((8, nb), jnp.float32),
            pltpu.VMEM((8, nb), jnp.float32),
            pltpu.SemaphoreType.DMA,
            pltpu.SemaphoreType.DMA,
            pltpu.SemaphoreType.DMA,
            pltpu.SemaphoreType.DMA,
            pltpu.SemaphoreType.DMA,
        ],
    )
    def k(idx_hbm, table_hbm, out_hbm, tbl_v, idx0, idx1, st0, st1,
          tsem, isem0, isem1, osem0, osem1):
        idxb = (idx0, idx1)
        stb = (st0, st1)
        isems = (isem0, isem1)
        osems = (osem0, osem1)
        wid = lax.axis_index("s") * _NC + lax.axis_index("c")
        dt = wid % n_dt
        hc = wid // n_dt
        d0 = pl.multiple_of(dt * 8, 8)

        ct = pltpu.async_copy(table_hbm, tbl_v, tsem)

        def h_of(j):
            return jnp.minimum(hc + 4 * j, hist - 1)

        def fire_idx(j):
            r = j % 2
            return pltpu.async_copy(
                idx_hbm.at[pl.ds(h_of(j) * nb, nb)], idxb[r], isems[r])

        ic = {0: fire_idx(0), 1: fire_idx(1)}
        ct.wait()

        o = [None, None]
        for j in range(n_hc):
            r = j % 2
            ic[j].wait()
            if o[r] is not None:
                o[r].wait()
            iv = idxb[r]
            st = stb[r]

            @plsc.parallel_loop(0, nb // _L, 1, unroll=2)
            def grp(g, iv=iv, st=st):
                bidx = iv[pl.ds(g * _L, _L)]
                addr = bidx * dim + d0
                for dsub in range(8):
                    st[dsub, pl.ds(g * _L, _L)] = plsc.load_gather(
                        tbl_v, [addr])
                    if dsub < 7:
                        addr = addr + 1
            if j + 2 < n_hc:
                ic[j + 2] = fire_idx(j + 2)
            o[r] = pltpu.async_copy(
                st, out_hbm.at[h_of(j), pl.ds(d0, 8)], osems[r])
        for r in range(2):
            if o[r] is not None:
                o[r].wait()

    return k(idx_flat, table_flat)


def kernel(tensor, weight):
    nb, hist = tensor.shape
    vocab, dim = weight.shape
    idx_flat = tensor.astype(jnp.int32).T.reshape(-1)
    table_flat = weight.reshape(-1)
    out = _gather_rows(idx_flat, table_flat, nb=nb, hist=hist, dim=dim)
    return out.transpose(2, 0, 1)
